# baseline (speedup 1.0000x reference)
import functools
import math

import jax
import jax.numpy as jnp
from jax import lax
from jax.experimental import pallas as pl
from jax.experimental.pallas import tpu as pltpu
from jax.experimental.pallas import tpu_sc as plsc

D_MODEL = 64
SCALE = math.sqrt(D_MODEL)
LANES = 16


@functools.lru_cache(maxsize=None)
def _build(B):
    info = plsc.get_sparse_core_info()
    NC, NS = info.num_cores, info.num_subcores
    NW = NC * NS
    assert B % NW == 0
    b_per_w = B // NW
    CH = 512
    assert b_per_w % CH == 0
    n_ch = b_per_w // CH

    mesh = plsc.VectorSubcoreMesh(core_axis_name="c", subcore_axis_name="s")

    @functools.partial(
        pl.kernel,
        mesh=mesh,
        out_type=jax.ShapeDtypeStruct((B, D_MODEL), jnp.float32),
        compiler_params=pltpu.CompilerParams(use_tc_tiling_on_sc=False),
        scratch_types=[
            pltpu.VMEM((b_per_w,), jnp.int32),
            pltpu.VMEM((CH, D_MODEL), jnp.float32),
            pltpu.SemaphoreType.DMA,
        ],
    )
    def emb(idx_hbm, table_hbm, out_hbm, idx_v, buf, sem):
        wid = lax.axis_index("s") * NC + lax.axis_index("c")
        base = wid * b_per_w
        pltpu.sync_copy(idx_hbm.at[pl.ds(base, b_per_w)], idx_v)

        def chunk(c, carry):
            off = c * CH
            pltpu.async_copy(
                table_hbm.at[idx_v.at[pl.ds(off, CH)]], buf, sem
            ).wait()

            def row(r, carry2):
                for j in range(D_MODEL // LANES):
                    sl = pl.ds(j * LANES, LANES)
                    buf[r, sl] = buf[r, sl] * SCALE
                return carry2

            lax.fori_loop(0, CH, row, 0)
            pltpu.sync_copy(buf, out_hbm.at[pl.ds(base + off, CH)])
            return carry

        lax.fori_loop(0, n_ch, chunk, 0)

    return emb


def kernel(x, table):
    B = x.shape[0] * x.shape[1]
    idx = x.reshape(-1).astype(jnp.int32)
    out = _build(B)(idx, table)
    return out.reshape(x.shape + (D_MODEL,))


# speedup vs baseline: 1.1134x; 1.1134x over previous
import functools
import math

import jax
import jax.numpy as jnp
from jax import lax
from jax.experimental import pallas as pl
from jax.experimental.pallas import tpu as pltpu
from jax.experimental.pallas import tpu_sc as plsc

D_MODEL = 64
SCALE = math.sqrt(D_MODEL)
LANES = 16
CH = 320
UNROLL = 4


@functools.lru_cache(maxsize=None)
def _build(B):
    info = plsc.get_sparse_core_info()
    NC, NS = info.num_cores, info.num_subcores
    NW = NC * NS
    assert B % NW == 0
    b_per_w = B // NW
    assert b_per_w % CH == 0
    n_ch = b_per_w // CH
    assert n_ch % 2 == 0 and n_ch >= 6
    assert CH % UNROLL == 0

    mesh = plsc.VectorSubcoreMesh(core_axis_name="c", subcore_axis_name="s")

    @functools.partial(
        pl.kernel,
        mesh=mesh,
        out_type=jax.ShapeDtypeStruct((B, D_MODEL), jnp.float32),
        compiler_params=pltpu.CompilerParams(use_tc_tiling_on_sc=False),
        scratch_types=[
            pltpu.VMEM((b_per_w,), jnp.int32),
            pltpu.VMEM((CH, D_MODEL), jnp.float32),
            pltpu.VMEM((CH, D_MODEL), jnp.float32),
            pltpu.VMEM((CH, D_MODEL), jnp.float32),
            pltpu.VMEM((CH, D_MODEL), jnp.float32),
            pltpu.SemaphoreType.DMA,
            pltpu.SemaphoreType.DMA,
            pltpu.SemaphoreType.DMA,
            pltpu.SemaphoreType.DMA,
        ],
    )
    def emb(idx_hbm, table_hbm, out_hbm, idx_v, gbuf0, gbuf1, obuf0, obuf1,
            gsem0, gsem1, osem0, osem1):
        gbuf = (gbuf0, gbuf1)
        obuf = (obuf0, obuf1)
        gsem = (gsem0, gsem1)
        osem = (osem0, osem1)

        wid = lax.axis_index("s") * NC + lax.axis_index("c")
        base = wid * b_per_w
        pltpu.sync_copy(idx_hbm.at[pl.ds(base, b_per_w)], idx_v)

        def issue_gather(c, b):
            pltpu.async_copy(
                table_hbm.at[idx_v.at[pl.ds(c * CH, CH)]], gbuf[b], gsem[b]
            )

        def wait_gather(c, b):
            pltpu.make_async_copy(
                table_hbm.at[idx_v.at[pl.ds(c * CH, CH)]], gbuf[b], gsem[b]
            ).wait()

        def issue_out(c, b):
            pltpu.async_copy(
                obuf[b], out_hbm.at[pl.ds(base + c * CH, CH)], osem[b]
            )

        def wait_out(c, b):
            pltpu.make_async_copy(
                obuf[b], out_hbm.at[pl.ds(base + c * CH, CH)], osem[b]
            ).wait()

        def scale(b):
            def rows(r4, carry):
                r = r4 * UNROLL
                for u in range(UNROLL):
                    for j in range(D_MODEL // LANES):
                        sl = pl.ds(j * LANES, LANES)
                        obuf[b][r + u, sl] = gbuf[b][r + u, sl] * SCALE
                return carry

            lax.fori_loop(0, CH // UNROLL, rows, 0)

        issue_gather(0, 0)
        issue_gather(1, 1)
        for b in range(2):
            wait_gather(b, b)
            scale(b)
            issue_out(b, b)
            issue_gather(b + 2, b)

        def group(i, carry):
            for b in range(2):
                c = 2 * i + b
                wait_gather(c, b)
                wait_out(c - 2, b)
                scale(b)
                issue_out(c, b)
                issue_gather(c + 2, b)
            return carry

        lax.fori_loop(1, n_ch // 2 - 1, group, 0)

        for b in range(2):
            c = n_ch - 2 + b
            wait_gather(c, b)
            wait_out(c - 2, b)
            scale(b)
            issue_out(c, b)
        for b in range(2):
            wait_out(n_ch - 2 + b, b)

    return emb


def kernel(x, table):
    B = x.shape[0] * x.shape[1]
    idx = x.reshape(-1).astype(jnp.int32)
    out = _build(B)(idx, table)
    return out.reshape(x.shape + (D_MODEL,))
